# baseline (speedup 1.0000x reference)
import functools

import jax
import jax.numpy as jnp
from jax import lax
from jax.experimental import pallas as pl
from jax.experimental.pallas import tpu as pltpu
from jax.experimental.pallas import tpu_sc as plsc

_N = 100000
_E = 1600000
_B = 256

_NT = 100096
_ACC = _NT + 8
_EPAD = 1605632
_ER = _EPAD // 128
_RPT = _NT // 16
_RNG = _NT // 8
_RNG_RPT = 782
_ROWS = 1000

_mesh = plsc.VectorSubcoreMesh(core_axis_name="c", subcore_axis_name="s")


def _zero_fill(zero_v):
    def zrow(i, _):
        zero_v[i] = jnp.zeros((16,), jnp.float32)
        return 0
    lax.fori_loop(0, zero_v.shape[0], zrow, 0)


def _zero_fill_wide(zero_v):
    def zrow(i, _):
        for t in range(8):
            zero_v[i, pl.ds(16 * t, 16)] = jnp.zeros((16,), jnp.float32)
        return 0
    lax.fori_loop(0, zero_v.shape[0], zrow, 0)


def _sweep_edges(src_ref, dst_ref, table, acc, src_v, dst_v, rows_v, gsem,
                 row_base, n_chunks):
    def chunk(k, _):
        r0 = row_base + k * 8
        pltpu.sync_copy(src_ref.at[pl.ds(r0, 8)], src_v)
        pltpu.sync_copy(dst_ref.at[pl.ds(r0, 8)], dst_v)
        cps = [pltpu.async_copy(table.at[src_v.at[j]],
                                rows_v.at[pl.ds(j * 128, 128)], gsem)
               for j in range(8)]
        for cp in cps:
            cp.wait()
        for j in range(8):
            pltpu.sync_copy(rows_v.at[pl.ds(j * 128, 128)],
                            acc.at[dst_v.at[j]], add=True)
        return 0
    lax.fori_loop(0, n_chunks, chunk, 0)


def _zero_slice(acc, zero_v, base):
    for j in range(_RPT // 1024):
        pltpu.sync_copy(zero_v, acc.at[pl.ds(base + j * 1024, 1024)])
    rem = _RPT % 1024
    if rem:
        pltpu.sync_copy(zero_v.at[pl.ds(0, rem)],
                        acc.at[pl.ds(base + _RPT - rem, rem)])


def _agg_body(src_ref, dst_ref, table, out,
              acc, src_v, dst_v, csrc, cdst, cdst2, rows_v, gsem):
    c = lax.axis_index("c")
    s = lax.axis_index("s")
    for p in range(4):
        base = (c * 4 + p) * _RNG
        _zero_fill_wide(rows_v)
        for j in range(6):
            pltpu.sync_copy(rows_v, acc.at[pl.ds(s * 782 + j * 128, 128)])
        pltpu.sync_copy(rows_v.at[pl.ds(0, 16)],
                        acc.at[pl.ds(s * 782 + 768, 16)])
        plsc.subcore_barrier()

        def chunk(k, _):
            r0 = s * (_ER // 16) + k * 8
            pltpu.sync_copy(src_ref.at[pl.ds(r0, 8)], src_v)
            pltpu.sync_copy(dst_ref.at[pl.ds(r0, 8)], dst_v)
            cnt = jnp.int32(0)
            for j in range(8):
                for t in range(8):
                    sv = src_v[j, pl.ds(16 * t, 16)]
                    dv = dst_v[j, pl.ds(16 * t, 16)] - base
                    m = (dv >= 0) & (dv < _RNG)
                    mi = jnp.where(m, 1, 0)
                    rank = plsc.cumsum(mi) - 1
                    lane = lax.iota(jnp.int32, 16)
                    pos = jnp.where(m, cnt + rank, 1152 + lane)
                    plsc.store_scatter(csrc, [pos], sv)
                    plsc.store_scatter(cdst, [pos], dv)
                    cnt = cnt + jnp.sum(mi)
            zi = jnp.zeros((16,), jnp.int32)
            ti = jnp.full((16,), _RNG, jnp.int32)
            for j in range(8):
                csrc[pl.ds(cnt + 16 * j, 16)] = zi
                cdst[pl.ds(cnt + 16 * j, 16)] = ti

            def blk(i, _):
                cp = pltpu.async_copy(table.at[csrc.at[pl.ds(i * 128, 128)]],
                                      rows_v, gsem)
                for t in range(8):
                    cdst2[0, pl.ds(16 * t, 16)] = cdst[pl.ds(i * 128 + 16 * t, 16)]
                cp.wait()
                pltpu.sync_copy(rows_v, acc.at[cdst2.at[0]], add=True)
                return 0
            lax.fori_loop(0, (cnt + 127) // 128, blk, 0)
            return 0
        lax.fori_loop(0, _ER // (16 * 8), chunk, 0)
        plsc.subcore_barrier()
        @pl.when(s < 15)
        def _():
            pltpu.sync_copy(acc.at[pl.ds(s * 784, 784)],
                            out.at[pl.ds(base + s * 784, 784)])

        @pl.when(s == 15)
        def _():
            pltpu.sync_copy(acc.at[pl.ds(11760, 752)],
                            out.at[pl.ds(base + 11760, 752)])
        plsc.subcore_barrier()


def _agg1_body(src_ref, dst_ref, x_ref, o0, o1,
               acc, src_v, dst_v, rows_v, gsem):
    c = lax.axis_index("c")
    s = lax.axis_index("s")
    base = s * _RPT
    _zero_fill(rows_v)
    _zero_slice(acc, rows_v, base)
    plsc.subcore_barrier()
    _sweep_edges(src_ref, dst_ref, x_ref, acc, src_v, dst_v, rows_v, gsem,
                 c * (_ER // 2) + s * (_ER // 32), _ER // (32 * 8))
    plsc.subcore_barrier()

    @pl.when(c == 0)
    def _():
        pltpu.sync_copy(acc.at[pl.ds(base, _RPT)], o0.at[pl.ds(base, _RPT)])

    @pl.when(c == 1)
    def _():
        pltpu.sync_copy(acc.at[pl.ds(base, _RPT)], o1.at[pl.ds(base, _RPT)])


_sc1_scratch = [
    pltpu.VMEM_SHARED((_ACC, 16), jnp.float32),
    pltpu.VMEM((8, 128), jnp.int32),
    pltpu.VMEM((8, 128), jnp.int32),
    pltpu.VMEM((1024, 16), jnp.float32),
    pltpu.SemaphoreType.DMA,
]

_agg_scratch = [
    pltpu.VMEM_SHARED((_RNG + 8, 128), jnp.float32),
    pltpu.VMEM((8, 128), jnp.int32),
    pltpu.VMEM((8, 128), jnp.int32),
    pltpu.VMEM((1168,), jnp.int32),
    pltpu.VMEM((1168,), jnp.int32),
    pltpu.VMEM((1, 128), jnp.int32),
    pltpu.VMEM((128, 128), jnp.float32),
    pltpu.SemaphoreType.DMA,
]

_agg = pl.kernel(
    _agg_body,
    out_type=jax.ShapeDtypeStruct((_NT, 128), jnp.float32),
    mesh=_mesh,
    scratch_types=_agg_scratch,
    compiler_params=pltpu.CompilerParams(needs_layout_passes=False),
)

_agg1 = pl.kernel(
    _agg1_body,
    out_type=tuple(jax.ShapeDtypeStruct((_NT, 16), jnp.float32)
                   for _ in range(2)),
    mesh=_mesh,
    scratch_types=_sc1_scratch,
    compiler_params=pltpu.CompilerParams(use_tc_tiling_on_sc=False),
)


def _norm_act(z, g_ref, be_ref):
    m = jnp.mean(z, axis=-1, keepdims=True)
    v = jnp.mean((z - m) ** 2, axis=-1, keepdims=True)
    zn = (z - m) * lax.rsqrt(v + 1e-5) * g_ref[...] + be_ref[...]
    return 0.5 * zn * (1.0 + lax.erf(zn * 0.7071067811865476))


def _dense1_body(p0_ref, p1_ref, x_ref, wl_ref, wr_ref, b_ref, g_ref, be_ref,
                 out_ref):
    agg = p0_ref[...] + p1_ref[...]
    cnt = agg[:, 15:16]
    mean = agg / jnp.maximum(cnt, 1.0)
    z = (jnp.dot(mean, wl_ref[...], preferred_element_type=jnp.float32)
         + jnp.dot(x_ref[...], wr_ref[...], preferred_element_type=jnp.float32)
         + b_ref[...])
    out_ref[...] = _norm_act(z, g_ref, be_ref)


def _dense_body(agg_ref, cnt_ref, h_ref, wl_ref, wr_ref, b_ref, g_ref, be_ref,
                out_ref):
    mean = agg_ref[...] / jnp.maximum(cnt_ref[...], 1.0)
    z = (jnp.dot(mean, wl_ref[...], preferred_element_type=jnp.float32)
         + jnp.dot(h_ref[...], wr_ref[...], preferred_element_type=jnp.float32)
         + b_ref[...])
    out_ref[...] = _norm_act(z, g_ref, be_ref)


def _dense4_body(agg_ref, cnt_ref, h_ref, wl_ref, wr_ref, b_ref, g_ref,
                 be_ref, batch_ref, pool_ref, pcnt_ref):
    mean = agg_ref[...] / jnp.maximum(cnt_ref[...], 1.0)
    z = (jnp.dot(mean, wl_ref[...], preferred_element_type=jnp.float32)
         + jnp.dot(h_ref[...], wr_ref[...], preferred_element_type=jnp.float32)
         + b_ref[...])
    h4 = _norm_act(z, g_ref, be_ref)
    oh = (batch_ref[...] == lax.broadcasted_iota(jnp.int32, (1, _B), 1)
          ).astype(jnp.float32)
    pool_blk = lax.dot_general(oh, h4, (((0,), (0,)), ((), ())),
                               preferred_element_type=jnp.float32)
    cnt_blk = jnp.sum(oh, axis=0, keepdims=True)

    @pl.when(pl.program_id(0) == 0)
    def _():
        pool_ref[...] = pool_blk
        pcnt_ref[...] = cnt_blk

    @pl.when(pl.program_id(0) > 0)
    def _():
        pool_ref[...] += pool_blk
        pcnt_ref[...] += cnt_blk


def _row_spec(d):
    return pl.BlockSpec((_ROWS, d), lambda i: (i, 0))


def _w_spec(d):
    return pl.BlockSpec((d, 128), lambda i: (0, 0))


_VEC_SPECS = [pl.BlockSpec((1, 128), lambda i: (0, 0))] * 3


def _dense1(p0, p1, x16, wl, wr, b, g, be):
    return pl.pallas_call(
        _dense1_body,
        grid=(_N // _ROWS,),
        in_specs=[_row_spec(16)] * 3 + [_w_spec(16)] * 2 + _VEC_SPECS,
        out_specs=_row_spec(128),
        out_shape=jax.ShapeDtypeStruct((_N, 128), jnp.float32),
    )(p0, p1, x16, wl, wr, b, g, be)


def _dense(agg, cnt, h, wl, wr, b, g, be):
    return pl.pallas_call(
        _dense_body,
        grid=(_N // _ROWS,),
        in_specs=[_row_spec(128), _row_spec(1), _row_spec(128)]
        + [_w_spec(128)] * 2 + _VEC_SPECS,
        out_specs=_row_spec(128),
        out_shape=jax.ShapeDtypeStruct((_N, 128), jnp.float32),
    )(agg, cnt, h, wl, wr, b, g, be)


def _dense4(agg, cnt, h, wl, wr, b, g, be, batch2d):
    return pl.pallas_call(
        _dense4_body,
        grid=(_N // _ROWS,),
        in_specs=[_row_spec(128), _row_spec(1), _row_spec(128)]
        + [_w_spec(128)] * 2 + _VEC_SPECS
        + [pl.BlockSpec((_ROWS, 1), lambda i: (i, 0))],
        out_specs=[pl.BlockSpec((_B, 128), lambda i: (0, 0)),
                   pl.BlockSpec((1, _B), lambda i: (0, 0))],
        out_shape=[jax.ShapeDtypeStruct((_B, 128), jnp.float32),
                   jax.ShapeDtypeStruct((1, _B), jnp.float32)],
    )(agg, cnt, h, wl, wr, b, g, be, batch2d)


def kernel(x, edge_index, batch,
           Wl1, Wr1, b1, g1, be1,
           Wl2, Wr2, b2, g2, be2,
           Wl3, Wr3, b3, g3, be3,
           Wl4, Wr4, b4, g4, be4):
    src = edge_index[0]
    dst = edge_index[1]
    npad = _EPAD - _E
    pad_ids = jnp.arange(npad, dtype=jnp.int32)
    src2d = jnp.concatenate([src, pad_ids % 128]).reshape(_ER, 128)
    dst2d = jnp.concatenate([dst, _NT + (pad_ids % 8)]).reshape(_ER, 128)

    x16 = jnp.concatenate([x, jnp.ones((_N, 1), jnp.float32)], axis=1)
    wl1p = jnp.pad(Wl1, ((0, 1), (0, 0)))
    wr1p = jnp.pad(Wr1, ((0, 1), (0, 0)))

    p0, p1 = _agg1(src2d, dst2d, x16)
    cnt = (p0[:_N, 15] + p1[:_N, 15])[:, None]
    h = _dense1(p0[:_N], p1[:_N], x16, wl1p, wr1p,
                b1[None, :], g1[None, :], be1[None, :])

    for wl, wr, b, g, be, last in ((Wl2, Wr2, b2, g2, be2, False),
                                   (Wl3, Wr3, b3, g3, be3, False),
                                   (Wl4, Wr4, b4, g4, be4, True)):
        agg = _agg(src2d, dst2d, h)[:_N]
        if not last:
            h = _dense(agg, cnt, h, wl, wr,
                       b[None, :], g[None, :], be[None, :])
        else:
            pooled, pcnt = _dense4(agg, cnt, h, wl, wr,
                                   b[None, :], g[None, :], be[None, :],
                                   batch[:, None])
    return pooled / jnp.clip(pcnt[0], 1.0, None)[:, None]


# speedup vs baseline: 9.9643x; 9.9643x over previous
import functools

import jax
import jax.numpy as jnp
from jax import lax
from jax.experimental import pallas as pl
from jax.experimental.pallas import tpu as pltpu
from jax.experimental.pallas import tpu_sc as plsc

_N = 100000
_E = 1600000
_B = 256

_NT = 100096
_ACC = _NT + 8
_EPAD = 1605632
_ER = _EPAD // 128
_RPT = _NT // 16
_ROWS = 1000

_mesh = plsc.VectorSubcoreMesh(core_axis_name="c", subcore_axis_name="s")


def _zero_fill(zero_v):
    def zrow(i, _):
        zero_v[i] = jnp.zeros((16,), jnp.float32)
        return 0
    lax.fori_loop(0, zero_v.shape[0], zrow, 0)


def _sweep_edges(src_ref, dst_ref, table, acc, src_v, dst_v, rows_v, gsem,
                 row_base, n_chunks):
    def load_idx(k, b):
        pltpu.sync_copy(src_ref.at[pl.ds(row_base + k * 4, 4)], src_v[b])
        pltpu.sync_copy(dst_ref.at[pl.ds(row_base + k * 4, 4)], dst_v[b])

    def fire(b):
        for j in range(4):
            pltpu.async_copy(table.at[src_v[b].at[j]],
                             rows_v[b].at[pl.ds(j * 128, 128)], gsem)

    def drain(b):
        for j in range(4):
            pltpu.make_async_copy(table.at[src_v[b].at[j]],
                                  rows_v[b].at[pl.ds(j * 128, 128)],
                                  gsem).wait()

    def scat(b):
        for j in range(4):
            pltpu.sync_copy(rows_v[b].at[pl.ds(j * 128, 128)],
                            acc.at[dst_v[b].at[j]], add=True)

    load_idx(0, 0)
    fire(0)

    def step(s2, _):
        for par in range(2):
            k = s2 * 2 + par

            @pl.when(k + 1 < n_chunks)
            def _(k=k, par=par):
                load_idx(k + 1, 1 - par)
            drain(par)

            @pl.when(k + 1 < n_chunks)
            def _(par=par):
                fire(1 - par)
            scat(par)
        return 0
    lax.fori_loop(0, n_chunks // 2, step, 0)


def _zero_slice(acc, zero_v, base):
    for j in range(_RPT // 1024):
        pltpu.sync_copy(zero_v[0], acc.at[pl.ds(base + j * 1024, 512)])
        pltpu.sync_copy(zero_v[1], acc.at[pl.ds(base + j * 1024 + 512, 512)])
    rem = _RPT % 1024
    if rem:
        pltpu.sync_copy(zero_v[0].at[pl.ds(0, rem)],
                        acc.at[pl.ds(base + _RPT - rem, rem)])


def _agg_body(src_ref, dst_ref, t0, t1, t2, t3, t4, t5, t6, t7,
              o0, o1, o2, o3, o4, o5, o6, o7,
              acc, src_v, dst_v, rows_v, gsem):
    c = lax.axis_index("c")
    s = lax.axis_index("s")
    tables = (t0, t1, t2, t3, t4, t5, t6, t7)
    outs = (o0, o1, o2, o3, o4, o5, o6, o7)
    base = s * _RPT
    for g in range(8):
        @pl.when(c == g // 4)
        def _(g=g):
            _zero_fill(rows_v[0])
            _zero_fill(rows_v[1])
            _zero_slice(acc, rows_v, base)
            plsc.subcore_barrier()
            _sweep_edges(src_ref, dst_ref, tables[g], acc,
                         src_v, dst_v, rows_v, gsem,
                         s * (_ER // 16), _ER // (16 * 4))
            plsc.subcore_barrier()
            pltpu.sync_copy(acc.at[pl.ds(base, _RPT)],
                            outs[g].at[pl.ds(base, _RPT)])
            plsc.subcore_barrier()


def _agg1_body(src_ref, dst_ref, x_ref, o0, o1,
               acc, src_v, dst_v, rows_v, gsem):
    c = lax.axis_index("c")
    s = lax.axis_index("s")
    base = s * _RPT
    _zero_fill(rows_v[0])
    _zero_fill(rows_v[1])
    _zero_slice(acc, rows_v, base)
    plsc.subcore_barrier()
    _sweep_edges(src_ref, dst_ref, x_ref, acc, src_v, dst_v, rows_v, gsem,
                 c * (_ER // 2) + s * (_ER // 32), _ER // (32 * 4))
    plsc.subcore_barrier()

    @pl.when(c == 0)
    def _():
        pltpu.sync_copy(acc.at[pl.ds(base, _RPT)], o0.at[pl.ds(base, _RPT)])

    @pl.when(c == 1)
    def _():
        pltpu.sync_copy(acc.at[pl.ds(base, _RPT)], o1.at[pl.ds(base, _RPT)])


_sc_scratch = [
    pltpu.VMEM_SHARED((_ACC, 16), jnp.float32),
    [pltpu.VMEM((4, 128), jnp.int32)] * 2,
    [pltpu.VMEM((4, 128), jnp.int32)] * 2,
    [pltpu.VMEM((512, 16), jnp.float32)] * 2,
    pltpu.SemaphoreType.DMA,
]

_sc_params = pltpu.CompilerParams(use_tc_tiling_on_sc=False)

_agg = pl.kernel(
    _agg_body,
    out_type=tuple(jax.ShapeDtypeStruct((_NT, 16), jnp.float32)
                   for _ in range(8)),
    mesh=_mesh,
    scratch_types=_sc_scratch,
    compiler_params=_sc_params,
)

_agg1 = pl.kernel(
    _agg1_body,
    out_type=tuple(jax.ShapeDtypeStruct((_NT, 16), jnp.float32)
                   for _ in range(2)),
    mesh=_mesh,
    scratch_types=_sc_scratch,
    compiler_params=_sc_params,
)


def _norm_act(z, g_ref, be_ref):
    m = jnp.mean(z, axis=-1, keepdims=True)
    v = jnp.mean((z - m) ** 2, axis=-1, keepdims=True)
    zn = (z - m) * lax.rsqrt(v + 1e-5) * g_ref[...] + be_ref[...]
    return 0.5 * zn * (1.0 + lax.erf(zn * 0.7071067811865476))


def _dense1_body(p0_ref, p1_ref, x_ref, wl_ref, wr_ref, b_ref, g_ref, be_ref,
                 out_ref):
    agg = p0_ref[...] + p1_ref[...]
    cnt = agg[:, 15:16]
    mean = agg / jnp.maximum(cnt, 1.0)
    z = (jnp.dot(mean, wl_ref[...], preferred_element_type=jnp.float32)
         + jnp.dot(x_ref[...], wr_ref[...], preferred_element_type=jnp.float32)
         + b_ref[...])
    out_ref[...] = _norm_act(z, g_ref, be_ref)


def _dense_body(agg_ref, cnt_ref, h_ref, wl_ref, wr_ref, b_ref, g_ref, be_ref,
                out_ref):
    mean = agg_ref[...] / jnp.maximum(cnt_ref[...], 1.0)
    z = (jnp.dot(mean, wl_ref[...], preferred_element_type=jnp.float32)
         + jnp.dot(h_ref[...], wr_ref[...], preferred_element_type=jnp.float32)
         + b_ref[...])
    out_ref[...] = _norm_act(z, g_ref, be_ref)


def _dense4_body(agg_ref, cnt_ref, h_ref, wl_ref, wr_ref, b_ref, g_ref,
                 be_ref, batch_ref, pool_ref, pcnt_ref):
    mean = agg_ref[...] / jnp.maximum(cnt_ref[...], 1.0)
    z = (jnp.dot(mean, wl_ref[...], preferred_element_type=jnp.float32)
         + jnp.dot(h_ref[...], wr_ref[...], preferred_element_type=jnp.float32)
         + b_ref[...])
    h4 = _norm_act(z, g_ref, be_ref)
    oh = (batch_ref[...] == lax.broadcasted_iota(jnp.int32, (1, _B), 1)
          ).astype(jnp.float32)
    pool_blk = lax.dot_general(oh, h4, (((0,), (0,)), ((), ())),
                               preferred_element_type=jnp.float32)
    cnt_blk = jnp.sum(oh, axis=0, keepdims=True)

    @pl.when(pl.program_id(0) == 0)
    def _():
        pool_ref[...] = pool_blk
        pcnt_ref[...] = cnt_blk

    @pl.when(pl.program_id(0) > 0)
    def _():
        pool_ref[...] += pool_blk
        pcnt_ref[...] += cnt_blk


def _row_spec(d):
    return pl.BlockSpec((_ROWS, d), lambda i: (i, 0))


def _w_spec(d):
    return pl.BlockSpec((d, 128), lambda i: (0, 0))


_VEC_SPECS = [pl.BlockSpec((1, 128), lambda i: (0, 0))] * 3


def _dense1(p0, p1, x16, wl, wr, b, g, be):
    return pl.pallas_call(
        _dense1_body,
        grid=(_N // _ROWS,),
        in_specs=[_row_spec(16)] * 3 + [_w_spec(16)] * 2 + _VEC_SPECS,
        out_specs=_row_spec(128),
        out_shape=jax.ShapeDtypeStruct((_N, 128), jnp.float32),
    )(p0, p1, x16, wl, wr, b, g, be)


def _dense(agg, cnt, h, wl, wr, b, g, be):
    return pl.pallas_call(
        _dense_body,
        grid=(_N // _ROWS,),
        in_specs=[_row_spec(128), _row_spec(1), _row_spec(128)]
        + [_w_spec(128)] * 2 + _VEC_SPECS,
        out_specs=_row_spec(128),
        out_shape=jax.ShapeDtypeStruct((_N, 128), jnp.float32),
    )(agg, cnt, h, wl, wr, b, g, be)


def _dense4(agg, cnt, h, wl, wr, b, g, be, batch2d):
    return pl.pallas_call(
        _dense4_body,
        grid=(_N // _ROWS,),
        in_specs=[_row_spec(128), _row_spec(1), _row_spec(128)]
        + [_w_spec(128)] * 2 + _VEC_SPECS
        + [pl.BlockSpec((_ROWS, 1), lambda i: (i, 0))],
        out_specs=[pl.BlockSpec((_B, 128), lambda i: (0, 0)),
                   pl.BlockSpec((1, _B), lambda i: (0, 0))],
        out_shape=[jax.ShapeDtypeStruct((_B, 128), jnp.float32),
                   jax.ShapeDtypeStruct((1, _B), jnp.float32)],
    )(agg, cnt, h, wl, wr, b, g, be, batch2d)


def kernel(x, edge_index, batch,
           Wl1, Wr1, b1, g1, be1,
           Wl2, Wr2, b2, g2, be2,
           Wl3, Wr3, b3, g3, be3,
           Wl4, Wr4, b4, g4, be4):
    src = edge_index[0]
    dst = edge_index[1]
    npad = _EPAD - _E
    pad_ids = jnp.arange(npad, dtype=jnp.int32)
    src2d = jnp.concatenate([src, pad_ids % 128]).reshape(_ER, 128)
    dst2d = jnp.concatenate([dst, _NT + (pad_ids % 8)]).reshape(_ER, 128)

    x16 = jnp.concatenate([x, jnp.ones((_N, 1), jnp.float32)], axis=1)
    wl1p = jnp.pad(Wl1, ((0, 1), (0, 0)))
    wr1p = jnp.pad(Wr1, ((0, 1), (0, 0)))

    p0, p1 = _agg1(src2d, dst2d, x16)
    cnt = (p0[:_N, 15] + p1[:_N, 15])[:, None]
    h = _dense1(p0[:_N], p1[:_N], x16, wl1p, wr1p,
                b1[None, :], g1[None, :], be1[None, :])

    for wl, wr, b, g, be, last in ((Wl2, Wr2, b2, g2, be2, False),
                                   (Wl3, Wr3, b3, g3, be3, False),
                                   (Wl4, Wr4, b4, g4, be4, True)):
        parts = _agg(src2d, dst2d,
                     *[h[:, 16 * j:16 * (j + 1)] for j in range(8)])
        agg = jnp.concatenate([o[:_N] for o in parts], axis=1)
        if not last:
            h = _dense(agg, cnt, h, wl, wr,
                       b[None, :], g[None, :], be[None, :])
        else:
            pooled, pcnt = _dense4(agg, cnt, h, wl, wr,
                                   b[None, :], g[None, :], be[None, :],
                                   batch[:, None])
    return pooled / jnp.clip(pcnt[0], 1.0, None)[:, None]


# speedup vs baseline: 10.4729x; 1.0510x over previous
import functools

import jax
import jax.numpy as jnp
from jax import lax
from jax.experimental import pallas as pl
from jax.experimental.pallas import tpu as pltpu
from jax.experimental.pallas import tpu_sc as plsc

_N = 100000
_E = 1600000
_B = 256

_NT = 100096
_ACC = _NT + 8
_EPAD = 1605632
_ER = _EPAD // 128
_RPT = _NT // 16
_ROWS = 1000

_mesh = plsc.VectorSubcoreMesh(core_axis_name="c", subcore_axis_name="s")


def _zero_fill(zero_v):
    def zrow(i, _):
        zero_v[i] = jnp.zeros((16,), jnp.float32)
        return 0
    lax.fori_loop(0, zero_v.shape[0], zrow, 0)


def _sweep_edges(src_ref, dst_ref, table, acc, src_v, dst_v, rows_v, gsem,
                 row_base, n_chunks):
    def load_idx(k, b):
        pltpu.sync_copy(src_ref.at[pl.ds(row_base + k * 4, 4)], src_v[b])
        pltpu.sync_copy(dst_ref.at[pl.ds(row_base + k * 4, 4)], dst_v[b])

    def fire(b):
        for j in range(4):
            pltpu.async_copy(table.at[src_v[b].at[j]],
                             rows_v[b].at[pl.ds(j * 128, 128)], gsem)

    def drain(b):
        for j in range(4):
            pltpu.make_async_copy(table.at[src_v[b].at[j]],
                                  rows_v[b].at[pl.ds(j * 128, 128)],
                                  gsem).wait()

    def scat(b):
        for j in range(4):
            pltpu.sync_copy(rows_v[b].at[pl.ds(j * 128, 128)],
                            acc.at[dst_v[b].at[j]], add=True)

    load_idx(0, 0)
    fire(0)

    def step(s2, _):
        for par in range(2):
            k = s2 * 2 + par

            @pl.when(k + 1 < n_chunks)
            def _(k=k, par=par):
                load_idx(k + 1, 1 - par)
            drain(par)

            @pl.when(k + 1 < n_chunks)
            def _(par=par):
                fire(1 - par)
            scat(par)
        return 0
    lax.fori_loop(0, n_chunks // 2, step, 0)


def _zero_slice(acc, zero_v, base):
    for j in range(_RPT // 1024):
        pltpu.sync_copy(zero_v[0], acc.at[pl.ds(base + j * 1024, 512)])
        pltpu.sync_copy(zero_v[1], acc.at[pl.ds(base + j * 1024 + 512, 512)])
    rem = _RPT % 1024
    if rem:
        pltpu.sync_copy(zero_v[0].at[pl.ds(0, rem)],
                        acc.at[pl.ds(base + _RPT - rem, rem)])


def _agg_body(src_ref, dst_ref, t0, t1, t2, t3, t4, t5, t6, t7,
              o0, o1, o2, o3, o4, o5, o6, o7,
              acc, src_v, dst_v, rows_v, gsem):
    c = lax.axis_index("c")
    s = lax.axis_index("s")
    tables = (t0, t1, t2, t3, t4, t5, t6, t7)
    outs = (o0, o1, o2, o3, o4, o5, o6, o7)
    base = s * _RPT
    for g in range(8):
        @pl.when(c == g // 4)
        def _(g=g):
            _zero_fill(rows_v[0])
            _zero_fill(rows_v[1])
            _zero_slice(acc, rows_v, base)
            plsc.subcore_barrier()
            _sweep_edges(src_ref, dst_ref, tables[g], acc,
                         src_v, dst_v, rows_v, gsem,
                         s * (_ER // 16), _ER // (16 * 4))
            plsc.subcore_barrier()
            pltpu.sync_copy(acc.at[pl.ds(base, _RPT)],
                            outs[g].at[pl.ds(base, _RPT)])
            plsc.subcore_barrier()


def _agg1_body(src_ref, dst_ref, x_ref, o0, o1,
               acc, src_v, dst_v, rows_v, gsem):
    c = lax.axis_index("c")
    s = lax.axis_index("s")
    base = s * _RPT
    _zero_fill(rows_v[0])
    _zero_fill(rows_v[1])
    _zero_slice(acc, rows_v, base)
    plsc.subcore_barrier()
    _sweep_edges(src_ref, dst_ref, x_ref, acc, src_v, dst_v, rows_v, gsem,
                 c * (_ER // 2) + s * (_ER // 32), _ER // (32 * 4))
    plsc.subcore_barrier()

    @pl.when(c == 0)
    def _():
        pltpu.sync_copy(acc.at[pl.ds(base, _RPT)], o0.at[pl.ds(base, _RPT)])

    @pl.when(c == 1)
    def _():
        pltpu.sync_copy(acc.at[pl.ds(base, _RPT)], o1.at[pl.ds(base, _RPT)])


_sc_scratch = [
    pltpu.VMEM_SHARED((_ACC, 16), jnp.float32),
    [pltpu.VMEM((4, 128), jnp.int32)] * 2,
    [pltpu.VMEM((4, 128), jnp.int32)] * 2,
    [pltpu.VMEM((512, 16), jnp.float32)] * 2,
    pltpu.SemaphoreType.DMA,
]

_sc_params = pltpu.CompilerParams(use_tc_tiling_on_sc=False)

_agg = pl.kernel(
    _agg_body,
    out_type=tuple(jax.ShapeDtypeStruct((_NT, 16), jnp.float32)
                   for _ in range(8)),
    mesh=_mesh,
    scratch_types=_sc_scratch,
    compiler_params=_sc_params,
)

_agg1 = pl.kernel(
    _agg1_body,
    out_type=tuple(jax.ShapeDtypeStruct((_NT, 16), jnp.float32)
                   for _ in range(2)),
    mesh=_mesh,
    scratch_types=_sc_scratch,
    compiler_params=_sc_params,
)


def _norm_act(z, g_ref, be_ref):
    m = jnp.mean(z, axis=-1, keepdims=True)
    v = jnp.mean((z - m) ** 2, axis=-1, keepdims=True)
    zn = (z - m) * lax.rsqrt(v + 1e-5) * g_ref[...] + be_ref[...]
    return 0.5 * zn * (1.0 + lax.erf(zn * 0.7071067811865476))


def _dense1_body(p0_ref, p1_ref, x_ref, wl_ref, wr_ref, b_ref, g_ref, be_ref,
                 out_ref):
    agg = p0_ref[...] + p1_ref[...]
    cnt = agg[:, 15:16]
    mean = agg / jnp.maximum(cnt, 1.0)
    z = (jnp.dot(mean, wl_ref[...], preferred_element_type=jnp.float32)
         + jnp.dot(x_ref[...], wr_ref[...], preferred_element_type=jnp.float32)
         + b_ref[...])
    out_ref[...] = _norm_act(z, g_ref, be_ref)


def _dense_body(p0, p1, p2, p3, p4, p5, p6, p7, cnt_ref, h_ref,
                wl_ref, wr_ref, b_ref, g_ref, be_ref, out_ref):
    agg = jnp.concatenate([p[...] for p in (p0, p1, p2, p3, p4, p5, p6, p7)],
                          axis=1)
    mean = agg / jnp.maximum(cnt_ref[...], 1.0)
    z = (jnp.dot(mean, wl_ref[...], preferred_element_type=jnp.float32)
         + jnp.dot(h_ref[...], wr_ref[...], preferred_element_type=jnp.float32)
         + b_ref[...])
    out_ref[...] = _norm_act(z, g_ref, be_ref)


def _dense4_body(p0, p1, p2, p3, p4, p5, p6, p7, cnt_ref, h_ref,
                 wl_ref, wr_ref, b_ref, g_ref, be_ref, batch_ref,
                 pool_ref, pcnt_ref):
    agg = jnp.concatenate([p[...] for p in (p0, p1, p2, p3, p4, p5, p6, p7)],
                          axis=1)
    mean = agg / jnp.maximum(cnt_ref[...], 1.0)
    z = (jnp.dot(mean, wl_ref[...], preferred_element_type=jnp.float32)
         + jnp.dot(h_ref[...], wr_ref[...], preferred_element_type=jnp.float32)
         + b_ref[...])
    h4 = _norm_act(z, g_ref, be_ref)
    oh = (batch_ref[...] == lax.broadcasted_iota(jnp.int32, (1, _B), 1)
          ).astype(jnp.float32)
    pool_blk = lax.dot_general(oh, h4, (((0,), (0,)), ((), ())),
                               preferred_element_type=jnp.float32)
    cnt_blk = jnp.sum(oh, axis=0, keepdims=True)

    @pl.when(pl.program_id(0) == 0)
    def _():
        pool_ref[...] = pool_blk
        pcnt_ref[...] = cnt_blk

    @pl.when(pl.program_id(0) > 0)
    def _():
        pool_ref[...] += pool_blk
        pcnt_ref[...] += cnt_blk


def _row_spec(d):
    return pl.BlockSpec((_ROWS, d), lambda i: (i, 0))


def _w_spec(d):
    return pl.BlockSpec((d, 128), lambda i: (0, 0))


_VEC_SPECS = [pl.BlockSpec((1, 128), lambda i: (0, 0))] * 3


def _dense1(p0, p1, x16, wl, wr, b, g, be):
    return pl.pallas_call(
        _dense1_body,
        grid=(_N // _ROWS,),
        in_specs=[_row_spec(16)] * 3 + [_w_spec(16)] * 2 + _VEC_SPECS,
        out_specs=_row_spec(128),
        out_shape=jax.ShapeDtypeStruct((_N, 128), jnp.float32),
    )(p0, p1, x16, wl, wr, b, g, be)


def _dense(parts, cnt, h, wl, wr, b, g, be):
    return pl.pallas_call(
        _dense_body,
        grid=(_N // _ROWS,),
        in_specs=[_row_spec(16)] * 8 + [_row_spec(1), _row_spec(128)]
        + [_w_spec(128)] * 2 + _VEC_SPECS,
        out_specs=_row_spec(128),
        out_shape=jax.ShapeDtypeStruct((_N, 128), jnp.float32),
    )(*parts, cnt, h, wl, wr, b, g, be)


def _dense4(parts, cnt, h, wl, wr, b, g, be, batch2d):
    return pl.pallas_call(
        _dense4_body,
        grid=(_N // _ROWS,),
        in_specs=[_row_spec(16)] * 8 + [_row_spec(1), _row_spec(128)]
        + [_w_spec(128)] * 2 + _VEC_SPECS
        + [pl.BlockSpec((_ROWS, 1), lambda i: (i, 0))],
        out_specs=[pl.BlockSpec((_B, 128), lambda i: (0, 0)),
                   pl.BlockSpec((1, _B), lambda i: (0, 0))],
        out_shape=[jax.ShapeDtypeStruct((_B, 128), jnp.float32),
                   jax.ShapeDtypeStruct((1, _B), jnp.float32)],
    )(*parts, cnt, h, wl, wr, b, g, be, batch2d)


def kernel(x, edge_index, batch,
           Wl1, Wr1, b1, g1, be1,
           Wl2, Wr2, b2, g2, be2,
           Wl3, Wr3, b3, g3, be3,
           Wl4, Wr4, b4, g4, be4):
    src = edge_index[0]
    dst = edge_index[1]
    npad = _EPAD - _E
    pad_ids = jnp.arange(npad, dtype=jnp.int32)
    src2d = jnp.concatenate([src, pad_ids % 128]).reshape(_ER, 128)
    dst2d = jnp.concatenate([dst, _NT + (pad_ids % 8)]).reshape(_ER, 128)

    x16 = jnp.concatenate([x, jnp.ones((_N, 1), jnp.float32)], axis=1)
    wl1p = jnp.pad(Wl1, ((0, 1), (0, 0)))
    wr1p = jnp.pad(Wr1, ((0, 1), (0, 0)))

    p0, p1 = _agg1(src2d, dst2d, x16)
    cnt = (p0[:_N, 15] + p1[:_N, 15])[:, None]
    h = _dense1(p0[:_N], p1[:_N], x16, wl1p, wr1p,
                b1[None, :], g1[None, :], be1[None, :])

    for wl, wr, b, g, be, last in ((Wl2, Wr2, b2, g2, be2, False),
                                   (Wl3, Wr3, b3, g3, be3, False),
                                   (Wl4, Wr4, b4, g4, be4, True)):
        parts = _agg(src2d, dst2d,
                     *[h[:, 16 * j:16 * (j + 1)] for j in range(8)])
        parts = [o[:_N] for o in parts]
        if not last:
            h = _dense(parts, cnt, h, wl, wr,
                       b[None, :], g[None, :], be[None, :])
        else:
            pooled, pcnt = _dense4(parts, cnt, h, wl, wr,
                                   b[None, :], g[None, :], be[None, :],
                                   batch[:, None])
    return pooled / jnp.clip(pcnt[0], 1.0, None)[:, None]
